# Initial kernel scaffold; baseline (speedup 1.0000x reference)
import jax
import jax.numpy as jnp
from jax.experimental import pallas as pl


def kernel(data, w_in, b_in, attn_in_w, attn_in_b, attn_out_w, attn_out_b, w_out, b_out):
    raise NotImplementedError("write your pallas kernel here")


# speedup vs baseline: 3.2349x; 3.2349x over previous
import functools

import jax
import jax.numpy as jnp
from jax import lax
from jax.experimental import pallas as pl
from jax.experimental.pallas import tpu as pltpu
from jax.experimental.pallas import tpu_sc as plsc

TOPK = 16
H = 8
DH = 16
E = 128
PAD_VAL = 1.0e9
BIG = 3.0e38


def _topk_body(rx_ref, ry_ref, cx_ref, cy_ref, out_ref):
    f = pl.program_id(0)
    rx = rx_ref[0]
    ry = ry_ref[0]
    cx = cx_ref[0]
    cy = cy_ref[0]
    rsq = rx * rx + ry * ry
    csq = cx * cx + cy * cy
    rxb = rx.astype(jnp.bfloat16).astype(jnp.float32)
    ryb = ry.astype(jnp.bfloat16).astype(jnp.float32)
    cxb = cx.astype(jnp.bfloat16).astype(jnp.float32)
    cyb = cy.astype(jnp.bfloat16).astype(jnp.float32)
    cross = rxb * cxb + ryb * cyb
    d2 = (rsq + csq) - 2.0 * cross
    d2 = jnp.sqrt(jnp.maximum(d2, 0.0))
    npad = d2.shape[1]
    lane = lax.broadcasted_iota(jnp.int32, (1, npad), 1)
    cols = []
    for _ in range(TOPK):
        m = jnp.min(d2, axis=1, keepdims=True)
        idx = jnp.min(jnp.where(d2 == m, lane, jnp.int32(npad)),
                      axis=1).astype(jnp.int32)
        cols.append(idx[:, None])
        d2 = jnp.where(lane == idx[:, None], BIG, d2)
    inds = jnp.concatenate(cols, axis=1)
    out_ref[0] = inds + f * jnp.int32(npad)


def _run_topk(xpad, ypad, nframes, npad, rblk):
    rx = xpad.reshape(nframes, npad, 1)
    ry = ypad.reshape(nframes, npad, 1)
    cx = xpad.reshape(nframes, 1, npad)
    cy = ypad.reshape(nframes, 1, npad)
    grid = (nframes, npad // rblk)
    out = pl.pallas_call(
        _topk_body,
        grid=grid,
        in_specs=[
            pl.BlockSpec((1, rblk, 1), lambda f, r: (f, r, 0)),
            pl.BlockSpec((1, rblk, 1), lambda f, r: (f, r, 0)),
            pl.BlockSpec((1, 1, npad), lambda f, r: (f, 0, 0)),
            pl.BlockSpec((1, 1, npad), lambda f, r: (f, 0, 0)),
        ],
        out_specs=pl.BlockSpec((1, rblk, TOPK), lambda f, r: (f, r, 0)),
        out_shape=jax.ShapeDtypeStruct((nframes, npad, TOPK), jnp.int32),
    )(rx, ry, cx, cy)
    return out


def _make_sc_gather(nrows_table, ncols, nidx):
    info = plsc.get_sparse_core_info()
    nw = info.num_cores * info.num_subcores
    per_w = nidx // nw
    chunk = 128
    nchunk = per_w // chunk
    mesh = plsc.VectorSubcoreMesh(core_axis_name="c", subcore_axis_name="s")

    @functools.partial(
        pl.kernel,
        mesh=mesh,
        out_type=jax.ShapeDtypeStruct((nidx, ncols), jnp.float32),
        scratch_types=[
            pltpu.VMEM((chunk,), jnp.int32),
            pltpu.VMEM((chunk, ncols), jnp.float32),
            pltpu.SemaphoreType.DMA,
        ],
        compiler_params=pltpu.CompilerParams(use_tc_tiling_on_sc=False),
    )
    def gather_k(table_hbm, idx_hbm, out_hbm, idx_v, rows_v, sem):
        wid = lax.axis_index("s") * info.num_cores + lax.axis_index("c")
        base = wid * per_w

        def body(i, _):
            off = base + i * chunk
            pltpu.sync_copy(idx_hbm.at[pl.ds(off, chunk)], idx_v)
            pltpu.async_copy(table_hbm.at[idx_v], rows_v, sem).wait()
            pltpu.sync_copy(rows_v, out_hbm.at[pl.ds(off, chunk)])
            return 0

        lax.fori_loop(0, nchunk, body, 0)

    return gather_k


def _feat_body(ctr_ref, nbr_ref, w_in_ref, b_in_ref, aiw_ref, aib_ref,
               aow_ref, aob_ref, w_out_ref, b_out_ref, out_ref):
    B = ctr_ref.shape[0]
    BS = B * TOPK

    ctr = ctr_ref[...]
    nbr16 = nbr_ref[...]

    def rep(col):
        return jnp.broadcast_to(col[:, None, :], (B, TOPK, 1)).reshape(BS, 1)

    cx = rep(ctr[:, 0:1])
    cy = rep(ctr[:, 1:2])
    ca = rep(ctr[:, 6:7])
    sin_a = jnp.sin(ca)
    cos_a = jnp.cos(ca)

    nx = nbr16[:, 0:1]
    ny = nbr16[:, 1:2]
    na = nbr16[:, 6:7]
    dx = nx - cx
    dy = ny - cy
    r2 = dx * dx + dy * dy
    edge_dist = jnp.sqrt(r2 + 1e-12) / 100.0
    inv = lax.rsqrt(r2)
    pos = r2 > 0.0
    s0 = jnp.where(pos, dy * inv, 0.0)
    c0 = jnp.where(pos, dx * inv, 1.0)
    sin_e = s0 * cos_a - c0 * sin_a
    cos_e = c0 * cos_a + s0 * sin_a
    ang = na - ca
    sin_n = jnp.sin(ang)
    cos_n = jnp.cos(ang)
    lwh = nbr16[:, 3:6] / 5.0

    feat = jnp.concatenate(
        [edge_dist, sin_e, cos_e, sin_n, cos_n, lwh], axis=1)

    x = jnp.maximum(jnp.dot(feat, w_in_ref[...],
                            preferred_element_type=jnp.float32)
                    + b_in_ref[...], 0.0)
    qkv = jnp.dot(x, aiw_ref[...],
                  preferred_element_type=jnp.float32) + aib_ref[...]
    q = qkv[:, :E] * 0.25
    k = qkv[:, E:2 * E]
    v = qkv[:, 2 * E:]

    q3 = q.reshape(B, TOPK, E)
    k3 = k.reshape(B, TOPK, E)
    v3 = v.reshape(B, TOPK, E)

    di = lax.broadcasted_iota(jnp.int32, (E, H), 0)
    hi = lax.broadcasted_iota(jnp.int32, (E, H), 1)
    g_head = (di // DH == hi).astype(jnp.float32)
    ei = lax.broadcasted_iota(jnp.int32, (H, E), 0)
    ci = lax.broadcasted_iota(jnp.int32, (H, E), 1)
    g_exp = (ci // DH == ei).astype(jnp.float32)
    ai = lax.broadcasted_iota(jnp.int32, (E, E), 0)
    bi = lax.broadcasted_iota(jnp.int32, (E, E), 1)
    g_mod = (ai % H == bi % H).astype(jnp.float32)

    sc = []
    for t in range(TOPK):
        prod = (q3 * k3[:, t:t + 1, :]).reshape(BS, E)
        sc.append(jnp.dot(prod, g_head,
                          preferred_element_type=jnp.float32))
    scores = jnp.concatenate(sc, axis=1)
    m = jnp.max(scores, axis=1, keepdims=True)
    ex = jnp.exp(scores - m)
    den = jnp.dot(ex, g_mod, preferred_element_type=jnp.float32)
    attn = ex / den

    o = jnp.zeros((B, TOPK, E), jnp.float32)
    for t in range(TOPK):
        ae = jnp.dot(attn[:, t * H:(t + 1) * H], g_exp,
                     preferred_element_type=jnp.float32)
        o = o + ae.reshape(B, TOPK, E) * v3[:, t:t + 1, :]

    o2 = jnp.dot(o.reshape(BS, E), aow_ref[...],
                 preferred_element_type=jnp.float32) + aob_ref[...]
    o3 = o2.reshape(B, TOPK, E)
    pooled = jnp.concatenate(
        [jnp.mean(o3, axis=1), jnp.max(o3, axis=1)], axis=1)
    out_ref[...] = jnp.dot(pooled, w_out_ref[...],
                           preferred_element_type=jnp.float32) + b_out_ref[...]


def _run_feat(ctr, nbrs, w_inT, b_in, aiwT, aib, aowT, aob, w_outT, b_out,
              bblk):
    nt = ctr.shape[0]
    grid = (nt // bblk,)
    full = lambda shape: pl.BlockSpec(shape, lambda i: tuple(0 for _ in shape))
    out = pl.pallas_call(
        _feat_body,
        grid=grid,
        in_specs=[
            pl.BlockSpec((bblk, 16), lambda i: (i, 0)),
            pl.BlockSpec((bblk * TOPK, 16), lambda i: (i, 0)),
            full((8, 128)),
            full((1, 128)),
            full((128, 384)),
            full((1, 384)),
            full((128, 128)),
            full((1, 128)),
            full((256, 128)),
            full((1, 128)),
        ],
        out_specs=pl.BlockSpec((bblk, 128), lambda i: (i, 0)),
        out_shape=jax.ShapeDtypeStruct((nt, 128), jnp.float32),
    )(ctr, nbrs, w_inT, b_in, aiwT, aib, aowT, aob, w_outT, b_out)
    return out


def kernel(data, w_in, b_in, attn_in_w, attn_in_b, attn_out_w, attn_out_b,
           w_out, b_out):
    boxes = data[0]
    F, N, C = boxes.shape
    RBLK = 128
    NPAD = ((N + RBLK - 1) // RBLK) * RBLK
    BBLK = 256

    table = jnp.full((F, NPAD, 16), 0.0, jnp.float32)
    table = table.at[:, :N, :C].set(boxes)
    if NPAD > N:
        table = table.at[:, N:, 0].set(PAD_VAL)
        table = table.at[:, N:, 1].set(PAD_VAL)

    xpad = table[:, :, 0].reshape(F, NPAD)
    ypad = table[:, :, 1].reshape(F, NPAD)

    inds = _run_topk(xpad, ypad, F, NPAD, RBLK)

    flat_table = table.reshape(F * NPAD, 16)
    flat_idx = inds.reshape(F * NPAD * TOPK)
    gathered = _make_sc_gather(F * NPAD, 16, flat_idx.shape[0])(
        flat_table, flat_idx)

    feats = _run_feat(
        flat_table, gathered,
        w_in.T, b_in.reshape(1, 128),
        attn_in_w.T, attn_in_b.reshape(1, 384),
        attn_out_w.T, attn_out_b.reshape(1, 128),
        w_out.T, b_out.reshape(1, 128),
        BBLK)

    feats = feats.reshape(F, NPAD, 128)[:, :N, :]
    return boxes, feats
